# Initial kernel scaffold; baseline (speedup 1.0000x reference)
import jax
import jax.numpy as jnp
from jax.experimental import pallas as pl


def kernel(encoded, raw_activations):
    raise NotImplementedError("write your pallas kernel here")


# speedup vs baseline: 24.2549x; 24.2549x over previous
import functools

import jax
import jax.numpy as jnp
from jax import lax
from jax.experimental import pallas as pl
from jax.experimental.pallas import tpu as pltpu

_N_PASS = 8


def _pool_body(enc_ref, raw_ref, out_ref, *, C, tho, W):
    TH = 2 * tho
    Wo = W // 2
    f32 = jnp.float32

    neg = jnp.full((TH, W), -jnp.inf, dtype=f32)

    def step(c, regs):
        carry = raw_ref[0, c]
        out = []
        for k in range(_N_PASS):
            r = regs[k]
            out.append(jnp.maximum(r, carry))
            if k + 1 < _N_PASS:
                carry = jnp.minimum(r, carry)
        return tuple(out)

    regs = lax.fori_loop(0, C, step, (neg,) * _N_PASS, unroll=4)
    lp = regs[_N_PASS - 1]

    iw = lax.broadcasted_iota(jnp.int32, (W, 2 * Wo), 0)
    jc = lax.broadcasted_iota(jnp.int32, (W, 2 * Wo), 1)
    tgt = jnp.where(jc < Wo, 2 * jc, 2 * (jc - Wo) + 1)
    S = (iw == tgt).astype(f32)

    dot = functools.partial(
        jnp.dot, precision=lax.Precision.HIGHEST, preferred_element_type=f32
    )

    for i in range(tho):
        lpc = dot(lp[2 * i : 2 * i + 2, :], S)
        la0 = lpc[0:1, :Wo]
        la1 = lpc[0:1, Wo:]
        lb0 = lpc[1:2, :Wo]
        lb1 = lpc[1:2, Wo:]
        m = jnp.maximum(jnp.maximum(la0, la1), jnp.maximum(lb0, lb1))

        ea = dot(enc_ref[0, :, 2 * i, :], S)
        eb = dot(enc_ref[0, :, 2 * i + 1, :], S)
        out = jnp.where(
            la0 == m,
            ea[:, :Wo],
            jnp.where(la1 == m, ea[:, Wo:], jnp.where(lb0 == m, eb[:, :Wo], eb[:, Wo:])),
        )
        out_ref[0, :, i, :] = out


def kernel(encoded, raw_activations):
    B, C, H, W = encoded.shape
    if H % 2 or W % 2:
        encoded = jnp.pad(encoded, ((0, 0), (0, 0), (0, H % 2), (0, W % 2)))
        raw_activations = jnp.pad(
            raw_activations, ((0, 0), (0, 0), (0, H % 2), (0, W % 2))
        )
        H += H % 2
        W += W % 2
    Ho, Wo = H // 2, W // 2
    tho = 8
    while Ho % tho:
        tho //= 2

    body = functools.partial(_pool_body, C=C, tho=tho, W=W)
    return pl.pallas_call(
        body,
        grid=(B, Ho // tho),
        in_specs=[
            pl.BlockSpec((1, C, 2 * tho, W), lambda b, j: (b, 0, j, 0)),
            pl.BlockSpec((1, C, 2 * tho, W), lambda b, j: (b, 0, j, 0)),
        ],
        out_specs=pl.BlockSpec((1, C, tho, Wo), lambda b, j: (b, 0, j, 0)),
        out_shape=jax.ShapeDtypeStruct((B, C, Ho, Wo), jnp.float32),
        compiler_params=pltpu.CompilerParams(
            dimension_semantics=("parallel", "parallel"),
        ),
    )(encoded, raw_activations)


# speedup vs baseline: 24.5514x; 1.0122x over previous
import functools

import jax
import jax.numpy as jnp
from jax import lax
from jax.experimental import pallas as pl
from jax.experimental.pallas import tpu as pltpu

_N_PASS = 8


def _pool_body(enc_ref, raw_ref, out_ref, *, C, n_out_rows, L):
    TH = 2 * n_out_rows
    Lo = L // 2
    f32 = jnp.float32

    neg = jnp.full((TH, L), -jnp.inf, dtype=f32)

    def step(c, regs):
        carry = raw_ref[0, c]
        out = []
        for k in range(_N_PASS):
            r = regs[k]
            out.append(jnp.maximum(r, carry))
            if k + 1 < _N_PASS:
                carry = jnp.minimum(r, carry)
        return tuple(out)

    regs = lax.fori_loop(0, C, step, (neg,) * _N_PASS, unroll=4)
    lp = regs[_N_PASS - 1]

    perm1 = jnp.concatenate(
        [jnp.arange(0, L, 2, dtype=jnp.int32), jnp.arange(1, L, 2, dtype=jnp.int32)]
    )[None, :]

    def deinterleave(x):
        p = jnp.take_along_axis(
            x,
            jnp.broadcast_to(perm1, x.shape),
            axis=1,
            mode="promise_in_bounds",
        )
        return p[:, :Lo], p[:, Lo:]

    lp0, lp1 = deinterleave(lp)

    for r in range(n_out_rows):
        ra = 4 * (r // 2) + (r % 2)
        rb = ra + 2
        a0 = lp0[ra : ra + 1, :]
        a1 = lp1[ra : ra + 1, :]
        b0 = lp0[rb : rb + 1, :]
        b1 = lp1[rb : rb + 1, :]
        m = jnp.maximum(jnp.maximum(a0, a1), jnp.maximum(b0, b1))

        ea0, ea1 = deinterleave(enc_ref[0, :, ra, :])
        eb0, eb1 = deinterleave(enc_ref[0, :, rb, :])
        out = jnp.where(
            a0 == m,
            ea0,
            jnp.where(a1 == m, ea1, jnp.where(b0 == m, eb0, eb1)),
        )
        out_ref[0, :, r, :] = out


def kernel(encoded, raw_activations):
    B, C, H, W = encoded.shape
    if H % 2 or W % 2:
        encoded = jnp.pad(encoded, ((0, 0), (0, 0), (0, H % 2), (0, W % 2)))
        raw_activations = jnp.pad(
            raw_activations, ((0, 0), (0, 0), (0, H % 2), (0, W % 2))
        )
        H += H % 2
        W += W % 2
    Ho, Wo = H // 2, W // 2

    if W % 4:
        encoded = jnp.pad(encoded, ((0, 0), (0, 0), (0, 0), (0, 2)))
        raw_activations = jnp.pad(
            raw_activations,
            ((0, 0), (0, 0), (0, 0), (0, 2)),
            constant_values=-jnp.inf,
        )
        W += 2

    L = W // 2
    enc_v = encoded.reshape(B, C, 2 * H, L)
    raw_v = raw_activations.reshape(B, C, 2 * H, L)

    n_out_rows = 8
    while (2 * Ho) % n_out_rows:
        n_out_rows //= 2

    body = functools.partial(_pool_body, C=C, n_out_rows=n_out_rows, L=L)
    out = pl.pallas_call(
        body,
        grid=(B, (2 * Ho) // n_out_rows),
        in_specs=[
            pl.BlockSpec((1, C, 2 * n_out_rows, L), lambda b, j: (b, 0, j, 0)),
            pl.BlockSpec((1, C, 2 * n_out_rows, L), lambda b, j: (b, 0, j, 0)),
        ],
        out_specs=pl.BlockSpec((1, C, n_out_rows, L // 2), lambda b, j: (b, 0, j, 0)),
        out_shape=jax.ShapeDtypeStruct((B, C, 2 * Ho, L // 2), jnp.float32),
        compiler_params=pltpu.CompilerParams(
            dimension_semantics=("parallel", "parallel"),
        ),
    )(enc_v, raw_v)
    out = out.reshape(B, C, Ho, W // 2)
    return out[:, :, :, :Wo]


# speedup vs baseline: 26.7041x; 1.0877x over previous
import functools

import jax
import jax.numpy as jnp
from jax import lax
from jax.experimental import pallas as pl
from jax.experimental.pallas import tpu as pltpu

_N_PASS = 8


def _pool_body(enc_ref, raw_ref, out_ref, *, C, n_out_rows, L):
    TH = 2 * n_out_rows
    Lo = L // 2
    f32 = jnp.float32

    def _ce(v, i, j):
        hi = jnp.maximum(v[i], v[j])
        lo = jnp.minimum(v[i], v[j])
        v[i], v[j] = hi, lo

    def _sort8(v):
        for i, j in (
            (0, 1), (2, 3), (4, 5), (6, 7),
            (0, 2), (1, 3), (4, 6), (5, 7),
            (1, 2), (5, 6),
            (0, 4), (1, 5), (2, 6), (3, 7),
            (2, 4), (3, 5),
            (1, 2), (3, 4), (5, 6),
        ):
            _ce(v, i, j)
        return v

    def _merge_top8(r, s):
        t = [jnp.maximum(r[i], s[7 - i]) for i in range(8)]
        for i, j in (
            (0, 4), (1, 5), (2, 6), (3, 7),
            (0, 2), (1, 3), (4, 6), (5, 7),
            (0, 1), (2, 3), (4, 5), (6, 7),
        ):
            _ce(t, i, j)
        return t

    n_groups = C // 8

    def group_vals(x):
        return _sort8([x[k] for k in range(8)])

    def step(g, regs):
        s = group_vals(raw_ref[0, pl.ds(g * 8, 8)])
        return tuple(_merge_top8(list(regs), s))

    regs = tuple(group_vals(raw_ref[0, pl.ds(0, 8)]))
    if n_groups > 1:
        regs = lax.fori_loop(1, n_groups, step, regs)
    regs = list(regs)
    for c in range(n_groups * 8, C):
        carry = raw_ref[0, c]
        for k in range(_N_PASS):
            r = regs[k]
            regs[k] = jnp.maximum(r, carry)
            if k + 1 < _N_PASS:
                carry = jnp.minimum(r, carry)
    lp = regs[_N_PASS - 1]

    perm1 = jnp.concatenate(
        [jnp.arange(0, L, 2, dtype=jnp.int32), jnp.arange(1, L, 2, dtype=jnp.int32)]
    )[None, :]

    def deinterleave(x):
        p = jnp.take_along_axis(
            x,
            jnp.broadcast_to(perm1, x.shape),
            axis=1,
            mode="promise_in_bounds",
        )
        return p[:, :Lo], p[:, Lo:]

    lp0, lp1 = deinterleave(lp)

    for r in range(n_out_rows):
        ra = 4 * (r // 2) + (r % 2)
        rb = ra + 2
        a0 = lp0[ra : ra + 1, :]
        a1 = lp1[ra : ra + 1, :]
        b0 = lp0[rb : rb + 1, :]
        b1 = lp1[rb : rb + 1, :]
        m = jnp.maximum(jnp.maximum(a0, a1), jnp.maximum(b0, b1))

        ea0, ea1 = deinterleave(enc_ref[0, :, ra, :])
        eb0, eb1 = deinterleave(enc_ref[0, :, rb, :])
        out = jnp.where(
            a0 == m,
            ea0,
            jnp.where(a1 == m, ea1, jnp.where(b0 == m, eb0, eb1)),
        )
        out_ref[0, :, r, :] = out


def kernel(encoded, raw_activations):
    B, C, H, W = encoded.shape
    if H % 2 or W % 2:
        encoded = jnp.pad(encoded, ((0, 0), (0, 0), (0, H % 2), (0, W % 2)))
        raw_activations = jnp.pad(
            raw_activations, ((0, 0), (0, 0), (0, H % 2), (0, W % 2))
        )
        H += H % 2
        W += W % 2
    Ho, Wo = H // 2, W // 2

    if W % 4:
        encoded = jnp.pad(encoded, ((0, 0), (0, 0), (0, 0), (0, 2)))
        raw_activations = jnp.pad(
            raw_activations,
            ((0, 0), (0, 0), (0, 0), (0, 2)),
            constant_values=-jnp.inf,
        )
        W += 2

    L = W // 2
    enc_v = encoded.reshape(B, C, 2 * H, L)
    raw_v = raw_activations.reshape(B, C, 2 * H, L)

    n_out_rows = 8
    while (2 * Ho) % n_out_rows:
        n_out_rows //= 2

    body = functools.partial(_pool_body, C=C, n_out_rows=n_out_rows, L=L)
    out = pl.pallas_call(
        body,
        grid=(B, (2 * Ho) // n_out_rows),
        in_specs=[
            pl.BlockSpec((1, C, 2 * n_out_rows, L), lambda b, j: (b, 0, j, 0)),
            pl.BlockSpec((1, C, 2 * n_out_rows, L), lambda b, j: (b, 0, j, 0)),
        ],
        out_specs=pl.BlockSpec((1, C, n_out_rows, L // 2), lambda b, j: (b, 0, j, 0)),
        out_shape=jax.ShapeDtypeStruct((B, C, 2 * Ho, L // 2), jnp.float32),
        compiler_params=pltpu.CompilerParams(
            dimension_semantics=("parallel", "parallel"),
        ),
    )(enc_v, raw_v)
    out = out.reshape(B, C, Ho, W // 2)
    return out[:, :, :, :Wo]


# speedup vs baseline: 27.1748x; 1.0176x over previous
import functools

import jax
import jax.numpy as jnp
from jax import lax
from jax.experimental import pallas as pl
from jax.experimental.pallas import tpu as pltpu

_N_PASS = 8

_SORT8_NET = (
    (0, 1), (2, 3), (4, 5), (6, 7),
    (0, 2), (1, 3), (4, 6), (5, 7),
    (1, 2), (5, 6),
    (0, 4), (1, 5), (2, 6), (3, 7),
    (2, 4), (3, 5),
    (1, 2), (3, 4), (5, 6),
)

_BITONIC8_NET = (
    (0, 4), (1, 5), (2, 6), (3, 7),
    (0, 2), (1, 3), (4, 6), (5, 7),
    (0, 1), (2, 3), (4, 5), (6, 7),
)


def _ce(v, i, j):
    hi = jnp.maximum(v[i], v[j])
    lo = jnp.minimum(v[i], v[j])
    v[i], v[j] = hi, lo


def _sort8(v):
    for i, j in _SORT8_NET:
        _ce(v, i, j)
    return v


def _merge_top8(r, s):
    t = [jnp.maximum(r[i], s[7 - i]) for i in range(8)]
    for i, j in _BITONIC8_NET:
        _ce(t, i, j)
    return t


def _lowest_passing(raw_ref, row0, rows, C):
    n_groups = C // 8

    def group_vals(x):
        return _sort8([x[k] for k in range(8)])

    def step(g, regs):
        s = group_vals(raw_ref[0, pl.ds(g * 8, 8), pl.ds(row0, rows), :])
        return tuple(_merge_top8(list(regs), s))

    regs = tuple(group_vals(raw_ref[0, pl.ds(0, 8), pl.ds(row0, rows), :]))
    if n_groups > 1:
        regs = lax.fori_loop(1, n_groups, step, regs)
    regs = list(regs)
    for c in range(n_groups * 8, C):
        carry = raw_ref[0, c, pl.ds(row0, rows), :]
        for k in range(_N_PASS):
            r = regs[k]
            regs[k] = jnp.maximum(r, carry)
            if k + 1 < _N_PASS:
                carry = jnp.minimum(r, carry)
    return regs[_N_PASS - 1]


def _pool_body(enc_ref, raw_ref, out_ref, *, C, n_out_rows, L):
    Lo = L // 2
    i32 = jnp.int32

    half_rows = 8 if n_out_rows >= 4 else 2 * n_out_rows
    out_rows_per_half = half_rows // 2
    n_halves = (2 * n_out_rows) // half_rows

    perm1 = jnp.concatenate(
        [jnp.arange(0, L, 2, dtype=i32), jnp.arange(1, L, 2, dtype=i32)]
    )[None, :]
    dup1 = (jnp.arange(L, dtype=i32) // 2)[None, :]
    lane1 = jnp.arange(L, dtype=i32)[None, :]

    def lane_gather(x, idx):
        return jnp.take_along_axis(
            x,
            jnp.broadcast_to(idx, x.shape),
            axis=1,
            mode="promise_in_bounds",
        )

    for h in range(n_halves):
        row0 = h * half_rows
        lp = _lowest_passing(raw_ref, row0, half_rows, C)
        p = lane_gather(lp, perm1)
        lp0, lp1 = p[:, :Lo], p[:, Lo:]

        tops, dws = [], []
        for r in range(out_rows_per_half):
            ra = 4 * (r // 2) + (r % 2)
            rb = ra + 2
            a0 = lp0[ra : ra + 1, :]
            a1 = lp1[ra : ra + 1, :]
            b0 = lp0[rb : rb + 1, :]
            b1 = lp1[rb : rb + 1, :]
            m = jnp.maximum(jnp.maximum(a0, a1), jnp.maximum(b0, b1))
            top = (a0 == m) | (a1 == m)
            dw = 1 - (jnp.where(top, a0, b0) == m).astype(i32)
            tops.append(top.astype(i32))
            dws.append(dw)
        top_s = jnp.concatenate(tops, axis=0)
        dw_s = jnp.concatenate(dws, axis=0)
        pad = jnp.zeros_like(top_s)
        top_exp = lane_gather(jnp.concatenate([top_s, pad], axis=1), dup1)
        dw_pad = jnp.concatenate([dw_s, pad], axis=1)
        lane_idx = jnp.where(lane1 < Lo, 2 * lane1 + dw_pad, 0)

        for r in range(out_rows_per_half):
            ra = row0 + 4 * (r // 2) + (r % 2)
            ea = enc_ref[0, :, ra, :]
            eb = enc_ref[0, :, ra + 2, :]
            g = jnp.where(top_exp[r : r + 1, :] > 0, ea, eb)
            out = lane_gather(g, lane_idx[r : r + 1, :])
            out_ref[0, :, h * out_rows_per_half + r, :] = out[:, :Lo]


def kernel(encoded, raw_activations):
    B, C, H, W = encoded.shape
    if H % 2 or W % 2:
        encoded = jnp.pad(encoded, ((0, 0), (0, 0), (0, H % 2), (0, W % 2)))
        raw_activations = jnp.pad(
            raw_activations, ((0, 0), (0, 0), (0, H % 2), (0, W % 2))
        )
        H += H % 2
        W += W % 2
    Ho, Wo = H // 2, W // 2

    if W % 4:
        encoded = jnp.pad(encoded, ((0, 0), (0, 0), (0, 0), (0, 2)))
        raw_activations = jnp.pad(
            raw_activations,
            ((0, 0), (0, 0), (0, 0), (0, 2)),
            constant_values=-jnp.inf,
        )
        W += 2

    L = W // 2
    enc_v = encoded.reshape(B, C, 2 * H, L)
    raw_v = raw_activations.reshape(B, C, 2 * H, L)

    n_out_rows = 8
    while (2 * Ho) % n_out_rows:
        n_out_rows //= 2

    body = functools.partial(_pool_body, C=C, n_out_rows=n_out_rows, L=L)
    out = pl.pallas_call(
        body,
        grid=(B, (2 * Ho) // n_out_rows),
        in_specs=[
            pl.BlockSpec((1, C, 2 * n_out_rows, L), lambda b, j: (b, 0, j, 0)),
            pl.BlockSpec((1, C, 2 * n_out_rows, L), lambda b, j: (b, 0, j, 0)),
        ],
        out_specs=pl.BlockSpec((1, C, n_out_rows, L // 2), lambda b, j: (b, 0, j, 0)),
        out_shape=jax.ShapeDtypeStruct((B, C, 2 * Ho, L // 2), jnp.float32),
        compiler_params=pltpu.CompilerParams(
            dimension_semantics=("parallel", "parallel"),
        ),
    )(enc_v, raw_v)
    out = out.reshape(B, C, Ho, W // 2)
    return out[:, :, :, :Wo]
